# Initial kernel scaffold; baseline (speedup 1.0000x reference)
import jax
import jax.numpy as jnp
from jax.experimental import pallas as pl


def kernel(x, edge_index, train_edge_weight, W1, b1, W2, b2):
    raise NotImplementedError("write your pallas kernel here")


# speedup vs baseline: 4.8217x; 4.8217x over previous
import functools

import jax
import jax.numpy as jnp
from jax import lax
from jax.experimental import pallas as pl
from jax.experimental.pallas import tpu as pltpu
from jax.experimental.pallas import tpu_sc as plsc

N = 50000
NP = 50176
E = 800000
EPAD = 819200
F_IN = 64
HID = 256
F_OUT = 128
K = 5

CW = 32
C = 512
RPT = NP // 16
WB = 112
PADR = 176

_mesh = lambda: plsc.VectorSubcoreMesh(core_axis_name="c", subcore_axis_name="s")
_f32 = jnp.float32
_i32 = jnp.int32


def _zero_vmem_2d(buf, rows):
    z = jnp.zeros((16,), _f32)

    def body(i, _):
        buf[i, pl.ds(0, 16)] = z
        buf[i, pl.ds(16, 16)] = z
        return 0

    lax.fori_loop(0, rows, body, 0)


def _zero_vmem_1d(buf, n):
    z = jnp.zeros((16,), _f32)

    def body(i, _):
        buf[pl.ds(i * 16, 16)] = z
        return 0

    lax.fori_loop(0, n // 16, body, 0)


def _sc_deg(row, ew):
    ept = EPAD // 32
    nchunk = ept // C
    stripe = NP // 16

    @functools.partial(
        pl.kernel,
        mesh=_mesh(),
        compiler_params=pltpu.CompilerParams(needs_layout_passes=False, use_tc_tiling_on_sc=False),
        out_type=jax.ShapeDtypeStruct((2 * NP,), _f32),
        scratch_types=[
            pltpu.VMEM_SHARED((NP,), _f32),
            pltpu.VMEM((stripe,), _f32),
            pltpu.VMEM((C,), _i32),
            pltpu.VMEM((C,), _f32),
        ],
    )
    def k(row_hbm, ew_hbm, deg_hbm, acc, wbuf, rowb, ewb):
        c = lax.axis_index("c")
        s = lax.axis_index("s")
        _zero_vmem_1d(wbuf, stripe)
        pltpu.sync_copy(wbuf, acc.at[pl.ds(s * stripe, stripe)])
        plsc.subcore_barrier()

        e0 = (c * 16 + s) * ept

        def chunk(i, _):
            eo = e0 + i * C
            pltpu.sync_copy(row_hbm.at[pl.ds(eo, C)], rowb)
            pltpu.sync_copy(ew_hbm.at[pl.ds(eo, C)], ewb)
            pltpu.sync_copy(ewb, acc.at[rowb], add=True)
            return 0

        lax.fori_loop(0, nchunk, chunk, 0)
        plsc.subcore_barrier()
        pltpu.sync_copy(acc.at[pl.ds(s * stripe, stripe)], wbuf)
        pltpu.sync_copy(wbuf, deg_hbm.at[pl.ds(c * NP + s * stripe, stripe)])

    return k(row, ew)


def _tc_dis(degp):
    def body(d_ref, o_ref):
        d = d_ref[0] + d_ref[1]
        o_ref[...] = jnp.where(d > 0, lax.rsqrt(jnp.where(d > 0, d, 1.0)), 0.0)

    return pl.pallas_call(
        body,
        out_shape=jax.ShapeDtypeStruct((392, 128), _f32),
    )(degp.reshape(2, 392, 128)).reshape(NP)


def _sc_norm(row, col, ew, dis):
    ept = EPAD // 32
    nchunk = ept // C

    @functools.partial(
        pl.kernel,
        mesh=_mesh(),
        compiler_params=pltpu.CompilerParams(needs_layout_passes=False, use_tc_tiling_on_sc=False),
        out_type=jax.ShapeDtypeStruct((EPAD,), _f32),
        scratch_types=[
            pltpu.VMEM((NP,), _f32),
            pltpu.VMEM((C,), _i32),
            pltpu.VMEM((C,), _i32),
            pltpu.VMEM((C,), _f32),
            pltpu.VMEM((C,), _f32),
        ],
    )
    def k(row_hbm, col_hbm, ew_hbm, dis_hbm, nrm_hbm, disb, rowb, colb, ewb, nrmb):
        c = lax.axis_index("c")
        s = lax.axis_index("s")
        pltpu.sync_copy(dis_hbm, disb)
        e0 = (c * 16 + s) * ept

        def chunk(i, _):
            eo = e0 + i * C
            pltpu.sync_copy(row_hbm.at[pl.ds(eo, C)], rowb)
            pltpu.sync_copy(col_hbm.at[pl.ds(eo, C)], colb)
            pltpu.sync_copy(ew_hbm.at[pl.ds(eo, C)], ewb)

            def vec(t, _):
                sl = pl.ds(t * 16, 16)
                dr = plsc.load_gather(disb, [rowb[sl]])
                dc = plsc.load_gather(disb, [colb[sl]])
                nrmb[sl] = (0.0 - dr) * ewb[sl] * dc
                return 0

            lax.fori_loop(0, C // 16, vec, 0)
            pltpu.sync_copy(nrmb, nrm_hbm.at[pl.ds(eo, C)])
            return 0

        lax.fori_loop(0, nchunk, chunk, 0)

    return k(row, col, ew, dis)


def _spmv_pass(table, row_hbm, col_hbm, nrm_hbm, acc, rowsbuf, rowb, colb,
               nrmb, sem, s):
    ept = EPAD // 16
    nchunk = ept // C

    def chunk(i, _):
        eo = s * ept + i * C
        pltpu.sync_copy(row_hbm.at[pl.ds(eo, C)], rowb)
        pltpu.sync_copy(col_hbm.at[pl.ds(eo, C)], colb)
        pltpu.sync_copy(nrm_hbm.at[pl.ds(eo, C)], nrmb)
        pltpu.async_copy(table.at[rowb], rowsbuf, sem).wait()

        def scale(t, _):
            for u in range(4):
                e = t * 4 + u
                spl = plsc.load_gather(nrmb, [jnp.full((16,), e, _i32)])
                rowsbuf[e, pl.ds(0, 16)] = rowsbuf[e, pl.ds(0, 16)] * spl
                rowsbuf[e, pl.ds(16, 16)] = rowsbuf[e, pl.ds(16, 16)] * spl
            return 0

        lax.fori_loop(0, C // 4, scale, 0)
        pltpu.sync_copy(rowsbuf, acc.at[colb], add=True)
        return 0

    lax.fori_loop(0, nchunk, chunk, 0)


def _writeback(acc, zbuf, sbuf, pbuf, out_at, s, fa, terms):
    r0 = s * RPT

    def sub(i, _):
        rr = r0 + i * WB
        pltpu.sync_copy(acc.at[pl.ds(rr, WB)], sbuf)
        pltpu.sync_copy(zbuf, acc.at[pl.ds(rr, WB)])
        first = True
        if fa == 2 and not terms:
            def dbl(r, _):
                for half in range(2):
                    sl = pl.ds(half * 16, 16)
                    v = sbuf[r, sl]
                    sbuf[r, sl] = v + v
                return 0

            lax.fori_loop(0, WB, dbl, 0)
        for ref_at, sign in terms:
            pltpu.sync_copy(ref_at.at[pl.ds(rr, WB)], pbuf)
            scale2 = fa == 2 and first
            first = False

            def comb(r, _, _scale2=scale2, _sign=sign):
                for half in range(2):
                    sl = pl.ds(half * 16, 16)
                    v = sbuf[r, sl]
                    if _scale2:
                        v = v + v
                    p = pbuf[r, sl]
                    sbuf[r, sl] = v + p if _sign > 0 else v - p
                return 0

            lax.fori_loop(0, WB, comb, 0)
        pltpu.sync_copy(sbuf, out_at.at[pl.ds(rr, WB)])
        return 0

    lax.fori_loop(0, RPT // WB, sub, 0)


_SC_SCRATCH = [
    pltpu.VMEM_SHARED((NP, CW), _f32),
    pltpu.VMEM((C, CW), _f32),
    pltpu.VMEM((C,), _i32),
    pltpu.VMEM((C,), _i32),
    pltpu.VMEM((C,), _f32),
    pltpu.VMEM((WB, CW), _f32),
    pltpu.VMEM((WB, CW), _f32),
    pltpu.VMEM((WB, CW), _f32),
    pltpu.SemaphoreType.DMA,
]


def _sc_cheb1(x_split, row, col, nrm):
    @functools.partial(
        pl.kernel,
        mesh=_mesh(),
        compiler_params=pltpu.CompilerParams(needs_layout_passes=False, use_tc_tiling_on_sc=False),
        out_type=jax.ShapeDtypeStruct((4, 2, NP, CW), _f32),
        scratch_types=_SC_SCRATCH,
    )
    def k(x_hbm, row_hbm, col_hbm, nrm_hbm, tk_hbm,
          acc, rowsbuf, rowb, colb, nrmb, zbuf, sbuf, pbuf, sem):
        c = lax.axis_index("c")
        s = lax.axis_index("s")
        _zero_vmem_2d(zbuf, WB)
        r0 = s * RPT
        for i in range(RPT // WB):
            pltpu.sync_copy(zbuf, acc.at[pl.ds(r0 + i * WB, WB)])
        plsc.subcore_barrier()

        for kk in range(1, K):
            table = x_hbm.at[c] if kk == 1 else tk_hbm.at[kk - 2, c]
            _spmv_pass(table, row_hbm, col_hbm, nrm_hbm, acc, rowsbuf,
                       rowb, colb, nrmb, sem, s)
            plsc.subcore_barrier()
            out_at = tk_hbm.at[kk - 1, c]
            if kk == 1:
                _writeback(acc, zbuf, sbuf, pbuf, out_at, s, 1, ())
            else:
                prev = x_hbm.at[c] if kk == 2 else tk_hbm.at[kk - 3, c]
                _writeback(acc, zbuf, sbuf, pbuf, out_at, s, 2,
                           ((prev, -1),))
            plsc.subcore_barrier()

    return k(x_split, row, col, nrm)


def _sc_cheb2(a_all, row, col, nrm):
    sds = jax.ShapeDtypeStruct

    @functools.partial(
        pl.kernel,
        mesh=_mesh(),
        compiler_params=pltpu.CompilerParams(needs_layout_passes=False, use_tc_tiling_on_sc=False),
        out_type=(sds((4, NP, CW), _f32), sds((4, NP, CW), _f32),
                  sds((4, NP, CW), _f32)),
        scratch_types=_SC_SCRATCH,
    )
    def k(a_hbm, row_hbm, col_hbm, nrm_hbm, out_hbm, b0_hbm, b1_hbm,
          acc, rowsbuf, rowb, colb, nrmb, zbuf, sbuf, pbuf, sem):
        c = lax.axis_index("c")
        s = lax.axis_index("s")
        _zero_vmem_2d(zbuf, WB)
        r0 = s * RPT
        for i in range(RPT // WB):
            pltpu.sync_copy(zbuf, acc.at[pl.ds(r0 + i * WB, WB)])
        plsc.subcore_barrier()

        steps = (
            (lambda j: a_hbm.at[4, j], lambda j: b1_hbm.at[j], 2,
             lambda j: ((a_hbm.at[3, j], 1),)),
            (lambda j: b1_hbm.at[j], lambda j: b0_hbm.at[j], 2,
             lambda j: ((a_hbm.at[2, j], 1), (a_hbm.at[4, j], -1))),
            (lambda j: b0_hbm.at[j], lambda j: b1_hbm.at[j], 2,
             lambda j: ((a_hbm.at[1, j], 1), (b1_hbm.at[j], -1))),
            (lambda j: b1_hbm.at[j], lambda j: out_hbm.at[j], 1,
             lambda j: ((a_hbm.at[0, j], 1), (b0_hbm.at[j], -1))),
        )
        for table_fn, out_fn, fa, terms_fn in steps:
            for jl in range(2):
                j = 2 * c + jl
                _spmv_pass(table_fn(j), row_hbm, col_hbm, nrm_hbm, acc,
                           rowsbuf, rowb, colb, nrmb, sem, s)
                plsc.subcore_barrier()
                _writeback(acc, zbuf, sbuf, pbuf, out_fn(j), s, fa,
                           terms_fn(j))
                plsc.subcore_barrier()

    return k(a_all, row, col, nrm)[0]


def _tc_mm1(x_split, tk, w1r, b1r):
    B = 1024

    def body(x_ref, tk_ref, w_ref, b_ref, h_ref):
        acc = jnp.broadcast_to(b_ref[0], (B, HID))
        for kk in range(K):
            for cc in range(2):
                t = x_ref[cc] if kk == 0 else tk_ref[kk - 1, cc]
                acc = acc + jnp.dot(t, w_ref[kk, cc],
                                    preferred_element_type=_f32)
        acc = jnp.maximum(acc, 0.0)
        for cc in range(8):
            h_ref[cc] = acc[:, cc * 32:(cc + 1) * 32]

    return pl.pallas_call(
        body,
        grid=(NP // B,),
        in_specs=[
            pl.BlockSpec((2, B, 32), lambda n: (0, n, 0)),
            pl.BlockSpec((4, 2, B, 32), lambda n: (0, 0, n, 0)),
            pl.BlockSpec((K, 2, 32, HID), lambda n: (0, 0, 0, 0)),
            pl.BlockSpec((1, HID), lambda n: (0, 0)),
        ],
        out_specs=pl.BlockSpec((8, B, 32), lambda n: (0, n, 0)),
        out_shape=jax.ShapeDtypeStruct((8, NP, 32), _f32),
    )(x_split, tk, w1r, b1r)


def _tc_mm2(h, w2r, b2r):
    B = 1024

    def body(h_ref, w_ref, b_ref, a_ref):
        for kk in range(K):
            acc = jnp.zeros((B, F_OUT), _f32)
            for cc in range(8):
                acc = acc + jnp.dot(h_ref[cc], w_ref[kk, cc],
                                    preferred_element_type=_f32)
            if kk == 0:
                acc = acc + b_ref[0]
            for j in range(4):
                a_ref[kk, j] = acc[:, j * 32:(j + 1) * 32]

    return pl.pallas_call(
        body,
        grid=(NP // B,),
        in_specs=[
            pl.BlockSpec((8, B, 32), lambda n: (0, n, 0)),
            pl.BlockSpec((K, 8, 32, F_OUT), lambda n: (0, 0, 0, 0)),
            pl.BlockSpec((1, F_OUT), lambda n: (0, 0)),
        ],
        out_specs=pl.BlockSpec((K, 4, B, 32), lambda n: (0, 0, n, 0)),
        out_shape=jax.ShapeDtypeStruct((K, 4, NP, 32), _f32),
    )(h, w2r, b2r)


def kernel(x, edge_index, train_edge_weight, W1, b1, W2, b2):
    npad = EPAD - E
    pad_idx = (N + jnp.arange(npad, dtype=_i32) % PADR)
    row = jnp.concatenate([edge_index[0], pad_idx])
    col = jnp.concatenate([edge_index[1], pad_idx])
    ew = jnp.concatenate([train_edge_weight, jnp.zeros((npad,), _f32)])

    xp = jnp.pad(x, ((0, NP - N), (0, 0)))
    x_split = xp.reshape(NP, 2, 32).transpose(1, 0, 2)
    w1r = W1.reshape(K, 2, 32, HID)
    w2r = W2.reshape(K, 8, 32, F_OUT)

    degp = _sc_deg(row, ew)
    dis = _tc_dis(degp)
    nrm = _sc_norm(row, col, ew, dis)

    tk = _sc_cheb1(x_split, row, col, nrm)
    h = _tc_mm1(x_split, tk, w1r, b1.reshape(1, HID))
    a_all = _tc_mm2(h, w2r, b2.reshape(1, F_OUT))
    outs = _sc_cheb2(a_all, row, col, nrm)

    return outs.transpose(1, 0, 2).reshape(NP, F_OUT)[:N]
